# baseline (speedup 1.0000x reference)
import functools

import jax
import jax.numpy as jnp
from jax import lax
from jax.experimental import pallas as pl
from jax.experimental.pallas import tpu as pltpu
from jax.experimental.pallas import tpu_sc as plsc

SIZE = 1024
DIM = 64
N = 32 * 576
B = 32
RPB = 576

NW = 32
RPW = N // NW
CHUNK = 96
NCHUNK = RPW // CHUNK
L = 16
CROWS = SIZE // L


def _tc_body(zT_ref, cb_ref, cn_ref, idx_ref):
    zT = zT_ref[0]
    zT16 = zT.astype(jnp.bfloat16)
    cb16 = cb_ref[...].astype(jnp.bfloat16)
    scoresT = lax.dot_general(cb16, zT16, (((1,), (0,)), ((), ())),
                              preferred_element_type=jnp.float32)
    znT = jnp.sum(zT * zT, axis=0, keepdims=True)
    costT = (znT + cn_ref[...]) - 2.0 * scoresT
    idx = jnp.argmin(costT, axis=0).astype(jnp.int32)
    idx_ref[0, 0, :] = idx


def _tc_argmin(zT, codebook, cnorm, interpret=False):
    return pl.pallas_call(
        _tc_body,
        grid=(B,),
        in_specs=[
            pl.BlockSpec((1, DIM, RPB), lambda i: (i, 0, 0)),
            pl.BlockSpec((SIZE, DIM), lambda i: (0, 0)),
            pl.BlockSpec((SIZE, 1), lambda i: (0, 0)),
        ],
        out_specs=[
            pl.BlockSpec((1, 1, RPB), lambda i: (i, 0, 0)),
        ],
        out_shape=[
            jax.ShapeDtypeStruct((B, 1, RPB), jnp.int32),
        ],
        compiler_params=pltpu.CompilerParams(
            dimension_semantics=("arbitrary",)),
        interpret=interpret,
    )(zT, codebook, cnorm)


def _tc_perp_body(c_ref, perp_ref):
    c = c_ref[0] + c_ref[1]
    e = c[:, 0:1] * (1.0 / N)
    perp_ref[0, 0] = jnp.exp(-jnp.sum(e * jnp.log(e + 1e-10)))


def _tc_perplexity(counts2):
    return pl.pallas_call(
        _tc_perp_body,
        out_specs=pl.BlockSpec(memory_space=pltpu.SMEM),
        out_shape=jax.ShapeDtypeStruct((1, 1), jnp.float32),
    )(counts2)


def _sc_gather_build():
    mesh = plsc.VectorSubcoreMesh(core_axis_name="c", subcore_axis_name="s")

    @functools.partial(
        pl.kernel,
        mesh=mesh,
        out_type=(
            jax.ShapeDtypeStruct((NW, RPW, DIM), jnp.float32),
            jax.ShapeDtypeStruct((2, SIZE, L), jnp.float32),
        ),
        scratch_types=[
            pltpu.VMEM((NCHUNK, CHUNK), jnp.int32),
            pltpu.VMEM((RPW, DIM), jnp.float32),
            pltpu.VMEM((CROWS, L), jnp.float32),
            pltpu.VMEM((CHUNK, L), jnp.float32),
            pltpu.VMEM_SHARED((SIZE, L), jnp.float32),
            pltpu.SemaphoreType.DMA,
            pltpu.SemaphoreType.DMA,
        ],
        compiler_params=pltpu.CompilerParams(use_tc_tiling_on_sc=False),
    )
    def _sc_gather(cb_hbm, idx_hbm, out_hbm, cnt_hbm,
                   idx_v, rows_v, zeros_v, ones_v, shared, sem, sem2):
        cid = lax.axis_index("c")
        sid = lax.axis_index("s")
        wid = sid * 2 + cid
        pltpu.sync_copy(idx_hbm.at[wid], idx_v)
        copies = []
        for j in range(NCHUNK):
            copies.append(pltpu.async_copy(
                cb_hbm.at[idx_v.at[j]],
                rows_v.at[pl.ds(j * CHUNK, CHUNK)],
                sem))

        def _zero(j, _):
            zeros_v[j, :] = jnp.zeros((L,), jnp.float32)
            return 0
        lax.fori_loop(0, CROWS, _zero, 0)

        def _one(j, _):
            ones_v[j, :] = jnp.ones((L,), jnp.float32)
            return 0
        lax.fori_loop(0, CHUNK, _one, 0)

        pltpu.sync_copy(zeros_v, shared.at[pl.ds(sid * CROWS, CROWS)])
        plsc.subcore_barrier()

        for j in range(NCHUNK):
            pltpu.sync_copy(ones_v, shared.at[idx_v.at[j]], add=True)

        stores = []
        for j in range(NCHUNK):
            copies[j].wait()
            stores.append(pltpu.async_copy(
                rows_v.at[pl.ds(j * CHUNK, CHUNK)],
                out_hbm.at[wid, pl.ds(j * CHUNK, CHUNK)],
                sem2))

        plsc.subcore_barrier()

        @pl.when(sid == 0)
        def _emit_counts():
            pltpu.sync_copy(shared, cnt_hbm.at[cid])

        for s in stores:
            s.wait()

    return _sc_gather


def kernel(z_from_encoder, codebook, codebook_weight, flg_train):
    z = z_from_encoder
    zT = jnp.swapaxes(z, 1, 2)
    cnorm = jnp.sum(codebook ** 2, axis=1, keepdims=True)
    idx = _tc_argmin(zT, codebook, cnorm)[0]
    idx_sc = idx.reshape(NW, NCHUNK, CHUNK)
    z_q, counts2 = _sc_gather_build()(codebook, idx_sc)
    perp = _tc_perplexity(counts2)
    return (z_q, 0.0, perp[0, 0])


# speedup vs baseline: 1.0296x; 1.0296x over previous
import functools

import jax
import jax.numpy as jnp
from jax import lax
from jax.experimental import pallas as pl
from jax.experimental.pallas import tpu as pltpu
from jax.experimental.pallas import tpu_sc as plsc

SIZE = 1024
DIM = 64
N = 32 * 576
B = 32
RPB = 576

NW = 32
RPW = N // NW
CHUNK = 96
NCHUNK = RPW // CHUNK
L = 16
CROWS = SIZE // L


def _tc_body(zT_ref, cb_ref, cn_ref, idx_ref):
    zT = zT_ref[0]
    zT16 = zT.astype(jnp.bfloat16)
    cb16 = cb_ref[...].astype(jnp.bfloat16)
    scoresT = lax.dot_general(cb16, zT16, (((1,), (0,)), ((), ())),
                              preferred_element_type=jnp.float32)
    znT = jnp.sum(zT * zT, axis=0, keepdims=True)
    costT = (znT + cn_ref[...]) - 2.0 * scoresT
    idx = jnp.argmin(costT, axis=0).astype(jnp.int32)
    idx_ref[0, 0, :] = idx


def _tc_argmin(zT, codebook, cnorm, interpret=False):
    return pl.pallas_call(
        _tc_body,
        grid=(B,),
        in_specs=[
            pl.BlockSpec((1, DIM, RPB), lambda i: (i, 0, 0)),
            pl.BlockSpec((SIZE, DIM), lambda i: (0, 0)),
            pl.BlockSpec((SIZE, 1), lambda i: (0, 0)),
        ],
        out_specs=[
            pl.BlockSpec((1, 1, RPB), lambda i: (i, 0, 0)),
        ],
        out_shape=[
            jax.ShapeDtypeStruct((B, 1, RPB), jnp.int32),
        ],
        compiler_params=pltpu.CompilerParams(
            dimension_semantics=("arbitrary",)),
        interpret=interpret,
    )(zT, codebook, cnorm)


def _tc_perp_body(c_ref, perp_ref):
    c = c_ref[0] + c_ref[1]
    e = c[:, 0:1] * (1.0 / N)
    perp_ref[0, 0] = jnp.exp(-jnp.sum(e * jnp.log(e + 1e-10)))


def _tc_perplexity(counts2):
    return pl.pallas_call(
        _tc_perp_body,
        out_specs=pl.BlockSpec(memory_space=pltpu.SMEM),
        out_shape=jax.ShapeDtypeStruct((1, 1), jnp.float32),
    )(counts2)


def _sc_gather_build():
    mesh = plsc.VectorSubcoreMesh(core_axis_name="c", subcore_axis_name="s")

    @functools.partial(
        pl.kernel,
        mesh=mesh,
        out_type=(
            jax.ShapeDtypeStruct((NW, RPW, DIM), jnp.float32),
            jax.ShapeDtypeStruct((2, SIZE, L), jnp.float32),
        ),
        scratch_types=[
            pltpu.VMEM((NCHUNK, CHUNK), jnp.int32),
            pltpu.VMEM((RPW, DIM), jnp.float32),
            pltpu.VMEM((CROWS, L), jnp.float32),
            pltpu.VMEM((CHUNK, L), jnp.float32),
            pltpu.VMEM_SHARED((SIZE, L), jnp.float32),
            pltpu.SemaphoreType.DMA,
            pltpu.SemaphoreType.DMA,
        ],
        compiler_params=pltpu.CompilerParams(use_tc_tiling_on_sc=False),
    )
    def _sc_gather(cb_hbm, idx_hbm, out_hbm, cnt_hbm,
                   idx_v, rows_v, zeros_v, ones_v, shared, sem, sem2):
        cid = lax.axis_index("c")
        sid = lax.axis_index("s")
        wid = sid * 2 + cid
        pltpu.sync_copy(idx_hbm.at[wid], idx_v)
        copies = []
        for j in range(NCHUNK):
            copies.append(pltpu.async_copy(
                cb_hbm.at[idx_v.at[j]],
                rows_v.at[pl.ds(j * CHUNK, CHUNK)],
                sem))

        def _zero(j, _):
            zeros_v[j, :] = jnp.zeros((L,), jnp.float32)
            return 0
        lax.fori_loop(0, CROWS, _zero, 0)

        def _one(j, _):
            ones_v[j, :] = jnp.ones((L,), jnp.float32)
            return 0
        lax.fori_loop(0, CHUNK, _one, 0)

        pltpu.sync_copy(zeros_v, shared.at[pl.ds(sid * CROWS, CROWS)])
        plsc.subcore_barrier()

        for j in range(NCHUNK):
            pltpu.sync_copy(ones_v, shared.at[idx_v.at[j]], add=True)

        for c in copies:
            c.wait()
        st = pltpu.async_copy(rows_v, out_hbm.at[wid], sem2)

        plsc.subcore_barrier()

        @pl.when(sid == 0)
        def _emit_counts():
            pltpu.sync_copy(shared, cnt_hbm.at[cid])

        st.wait()

    return _sc_gather


def kernel(z_from_encoder, codebook, codebook_weight, flg_train):
    z = z_from_encoder
    zT = jnp.swapaxes(z, 1, 2)
    cnorm = jnp.sum(codebook ** 2, axis=1, keepdims=True)
    idx = _tc_argmin(zT, codebook, cnorm)[0]
    idx_sc = idx.reshape(NW, NCHUNK, CHUNK)
    z_q, counts2 = _sc_gather_build()(codebook, idx_sc)
    perp = _tc_perplexity(counts2)
    return (z_q, 0.0, perp[0, 0])


# speedup vs baseline: 1.0561x; 1.0257x over previous
import functools

import jax
import jax.numpy as jnp
from jax import lax
from jax.experimental import pallas as pl
from jax.experimental.pallas import tpu as pltpu
from jax.experimental.pallas import tpu_sc as plsc

SIZE = 1024
DIM = 64
N = 32 * 576
B = 32
RPB = 576

NW = 32
RPW = N // NW
CHUNK = 96
NCHUNK = RPW // CHUNK
L = 16
CROWS = SIZE // L


def _tc_body(zT_ref, cb_ref, cn_ref, idx_ref):
    zT = zT_ref[0]
    zT16 = zT.astype(jnp.bfloat16)
    cb16 = cb_ref[...].astype(jnp.bfloat16)
    scoresT = lax.dot_general(cb16, zT16, (((1,), (0,)), ((), ())),
                              preferred_element_type=jnp.float32)
    znT = jnp.sum(zT * zT, axis=0, keepdims=True)
    costT = (znT + cn_ref[...]) - 2.0 * scoresT
    idx = jnp.argmin(costT, axis=0).astype(jnp.int32)
    idx_ref[0, 0, :] = idx


def _tc_argmin(zT, codebook, cnorm, interpret=False):
    return pl.pallas_call(
        _tc_body,
        grid=(B,),
        in_specs=[
            pl.BlockSpec((1, DIM, RPB), lambda i: (i, 0, 0)),
            pl.BlockSpec((SIZE, DIM), lambda i: (0, 0)),
            pl.BlockSpec((SIZE, 1), lambda i: (0, 0)),
        ],
        out_specs=[
            pl.BlockSpec((1, 1, RPB), lambda i: (i, 0, 0)),
        ],
        out_shape=[
            jax.ShapeDtypeStruct((B, 1, RPB), jnp.int32),
        ],
        compiler_params=pltpu.CompilerParams(
            dimension_semantics=("arbitrary",)),
        interpret=interpret,
    )(zT, codebook, cnorm)


def _tc_perp_body(c_ref, perp_ref):
    c = c_ref[0] + c_ref[1]
    e = c * (1.0 / N)
    h = jnp.sum(e * jnp.log(e + 1e-10)) * (1.0 / L)
    perp_ref[0, 0] = jnp.exp(-h)


def _tc_perplexity(counts2):
    return pl.pallas_call(
        _tc_perp_body,
        out_specs=pl.BlockSpec(memory_space=pltpu.SMEM),
        out_shape=jax.ShapeDtypeStruct((1, 1), jnp.float32),
    )(counts2)


def _sc_gather_build():
    mesh = plsc.VectorSubcoreMesh(core_axis_name="c", subcore_axis_name="s")

    @functools.partial(
        pl.kernel,
        mesh=mesh,
        out_type=(
            jax.ShapeDtypeStruct((NW, RPW, DIM), jnp.float32),
            jax.ShapeDtypeStruct((2, SIZE, L), jnp.float32),
        ),
        scratch_types=[
            pltpu.VMEM((NCHUNK, CHUNK), jnp.int32),
            pltpu.VMEM((RPW, DIM), jnp.float32),
            pltpu.VMEM((CROWS, L), jnp.float32),
            pltpu.VMEM((CHUNK, L), jnp.float32),
            pltpu.VMEM_SHARED((SIZE, L), jnp.float32),
            pltpu.SemaphoreType.DMA,
            pltpu.SemaphoreType.DMA,
        ],
        compiler_params=pltpu.CompilerParams(use_tc_tiling_on_sc=False),
    )
    def _sc_gather(cb_hbm, idx_hbm, out_hbm, cnt_hbm,
                   idx_v, rows_v, zeros_v, ones_v, shared, sem, sem2):
        cid = lax.axis_index("c")
        sid = lax.axis_index("s")
        wid = sid * 2 + cid
        pltpu.sync_copy(idx_hbm.at[wid], idx_v)
        copies = []
        for j in range(NCHUNK):
            copies.append(pltpu.async_copy(
                cb_hbm.at[idx_v.at[j]],
                rows_v.at[pl.ds(j * CHUNK, CHUNK)],
                sem))

        def _zero(j, _):
            zeros_v[j, :] = jnp.zeros((L,), jnp.float32)
            return 0
        lax.fori_loop(0, CROWS, _zero, 0)

        def _one(j, _):
            ones_v[j, :] = jnp.ones((L,), jnp.float32)
            return 0
        lax.fori_loop(0, CHUNK, _one, 0)

        pltpu.sync_copy(zeros_v, shared.at[pl.ds(sid * CROWS, CROWS)])
        plsc.subcore_barrier()

        for j in range(NCHUNK):
            pltpu.sync_copy(ones_v, shared.at[idx_v.at[j]], add=True)

        for c in copies:
            c.wait()
        st = pltpu.async_copy(rows_v, out_hbm.at[wid], sem2)

        plsc.subcore_barrier()

        @pl.when(sid == 0)
        def _emit_counts():
            pltpu.sync_copy(shared, cnt_hbm.at[cid])

        st.wait()

    return _sc_gather


def kernel(z_from_encoder, codebook, codebook_weight, flg_train):
    z = z_from_encoder
    zT = jnp.swapaxes(z, 1, 2)
    cnorm = jnp.sum(codebook ** 2, axis=1, keepdims=True)
    idx = _tc_argmin(zT, codebook, cnorm)[0]
    idx_sc = idx.reshape(NW, NCHUNK, CHUNK)
    z_q, counts2 = _sc_gather_build()(codebook, idx_sc)
    perp = _tc_perplexity(counts2.reshape(2, 128, 128))
    return (z_q, 0.0, perp[0, 0])


# speedup vs baseline: 1.1297x; 1.0697x over previous
import functools

import jax
import jax.numpy as jnp
from jax import lax
from jax.experimental import pallas as pl
from jax.experimental.pallas import tpu as pltpu
from jax.experimental.pallas import tpu_sc as plsc

SIZE = 1024
DIM = 64
N = 32 * 576
B = 32
RPB = 576
BPS = 2

NW = 32
RPW = N // NW
CHUNK = 96
NCHUNK = RPW // CHUNK
L = 16
CROWS = SIZE // L


def _tc_body(zT_ref, cb_ref, cn_ref, idx_ref):
    cb16 = cb_ref[...].astype(jnp.bfloat16)
    for j in range(BPS):
        zT = zT_ref[j]
        zT16 = zT.astype(jnp.bfloat16)
        scoresT = lax.dot_general(cb16, zT16, (((1,), (0,)), ((), ())),
                                  preferred_element_type=jnp.float32)
        znT = jnp.sum(zT * zT, axis=0, keepdims=True)
        costT = (znT + cn_ref[...]) - 2.0 * scoresT
        idx = jnp.argmin(costT, axis=0).astype(jnp.int32)
        idx_ref[j, 0, :] = idx


def _tc_argmin(zT, codebook, cnorm, interpret=False):
    return pl.pallas_call(
        _tc_body,
        grid=(B // BPS,),
        in_specs=[
            pl.BlockSpec((BPS, DIM, RPB), lambda i: (i, 0, 0)),
            pl.BlockSpec((SIZE, DIM), lambda i: (0, 0)),
            pl.BlockSpec((SIZE, 1), lambda i: (0, 0)),
        ],
        out_specs=[
            pl.BlockSpec((BPS, 1, RPB), lambda i: (i, 0, 0)),
        ],
        out_shape=[
            jax.ShapeDtypeStruct((B, 1, RPB), jnp.int32),
        ],
        compiler_params=pltpu.CompilerParams(
            dimension_semantics=("arbitrary",)),
        interpret=interpret,
    )(zT, codebook, cnorm)


def _tc_perp_body(c_ref, perp_ref):
    c = c_ref[0] + c_ref[1]
    e = c * (1.0 / N)
    h = jnp.sum(e * jnp.log(e + 1e-10)) * (1.0 / L)
    perp_ref[0, 0] = jnp.exp(-h)


def _tc_perplexity(counts2):
    return pl.pallas_call(
        _tc_perp_body,
        out_specs=pl.BlockSpec(memory_space=pltpu.SMEM),
        out_shape=jax.ShapeDtypeStruct((1, 1), jnp.float32),
    )(counts2)


def _sc_gather_build():
    mesh = plsc.VectorSubcoreMesh(core_axis_name="c", subcore_axis_name="s")

    @functools.partial(
        pl.kernel,
        mesh=mesh,
        out_type=(
            jax.ShapeDtypeStruct((NW, RPW, DIM), jnp.float32),
            jax.ShapeDtypeStruct((2, SIZE, L), jnp.float32),
        ),
        scratch_types=[
            pltpu.VMEM((NCHUNK, CHUNK), jnp.int32),
            pltpu.VMEM((RPW, DIM), jnp.float32),
            pltpu.VMEM((CROWS, L), jnp.float32),
            pltpu.VMEM((CHUNK, L), jnp.float32),
            pltpu.VMEM_SHARED((SIZE, L), jnp.float32),
            pltpu.SemaphoreType.DMA,
            pltpu.SemaphoreType.DMA,
        ],
        compiler_params=pltpu.CompilerParams(use_tc_tiling_on_sc=False),
    )
    def _sc_gather(cb_hbm, idx_hbm, out_hbm, cnt_hbm,
                   idx_v, rows_v, zeros_v, ones_v, shared, sem, sem2):
        cid = lax.axis_index("c")
        sid = lax.axis_index("s")
        wid = sid * 2 + cid
        pltpu.sync_copy(idx_hbm.at[wid], idx_v)
        copies = []
        for j in range(NCHUNK):
            copies.append(pltpu.async_copy(
                cb_hbm.at[idx_v.at[j]],
                rows_v.at[pl.ds(j * CHUNK, CHUNK)],
                sem))

        def _zero(j, _):
            zeros_v[j, :] = jnp.zeros((L,), jnp.float32)
            return 0
        lax.fori_loop(0, CROWS, _zero, 0)

        def _one(j, _):
            ones_v[j, :] = jnp.ones((L,), jnp.float32)
            return 0
        lax.fori_loop(0, CHUNK, _one, 0)

        pltpu.sync_copy(zeros_v, shared.at[pl.ds(sid * CROWS, CROWS)])
        plsc.subcore_barrier()

        for j in range(NCHUNK):
            pltpu.sync_copy(ones_v, shared.at[idx_v.at[j]], add=True)

        for c in copies:
            c.wait()
        st = pltpu.async_copy(rows_v, out_hbm.at[wid], sem2)

        plsc.subcore_barrier()

        @pl.when(sid == 0)
        def _emit_counts():
            pltpu.sync_copy(shared, cnt_hbm.at[cid])

        st.wait()

    return _sc_gather


def kernel(z_from_encoder, codebook, codebook_weight, flg_train):
    z = z_from_encoder
    zT = jnp.swapaxes(z, 1, 2)
    cnorm = jnp.sum(codebook ** 2, axis=1, keepdims=True)
    idx = _tc_argmin(zT, codebook, cnorm)[0]
    idx_sc = idx.reshape(NW, NCHUNK, CHUNK)
    z_q, counts2 = _sc_gather_build()(codebook, idx_sc)
    perp = _tc_perplexity(counts2.reshape(2, 128, 128))
    return (z_q, 0.0, perp[0, 0])
